# Initial kernel scaffold; baseline (speedup 1.0000x reference)
import functools

import jax
import jax.numpy as jnp
from jax import lax
from jax.experimental import pallas as pl
from jax.experimental.pallas import tpu as pltpu
from jax.experimental.pallas import tpu_sc as plsc

_VOCAB = 100000
_L = 200
_D = 128
_B = 1024
_N = _B * _L
_NW = 32
_RPW = _N // _NW
_CH = 100
_NCH = _RPW // _CH
_EPS = 1e-12
_INV_D = 1.0 / _D


def _rsqrt(v):
    i = lax.bitcast_convert_type(v, jnp.int32)
    i = jnp.int32(0x5F3759DF) - lax.shift_right_logical(i, 1)
    y = lax.bitcast_convert_type(i, jnp.float32)
    for _ in range(3):
        y = y * (1.5 - 0.5 * v * y * y)
    return y


def _body(idx_hbm, wt_hbm, pt_hbm, g_hbm, b_hbm, out_hbm,
          idx_v, idx_buf, pos_v, g_v, b_v, rows_v, gsem):
    wid = lax.axis_index("s") * 2 + lax.axis_index("c")
    base = wid * _RPW

    pltpu.sync_copy(idx_hbm.at[wid], idx_v)
    pltpu.sync_copy(pt_hbm, pos_v)
    pltpu.sync_copy(g_hbm, g_v)
    pltpu.sync_copy(b_hbm, b_v)

    def chunk_body(k, carry):
        pltpu.sync_copy(idx_v.at[k], idx_buf)
        pltpu.async_copy(wt_hbm.at[idx_buf], rows_v, gsem).wait()
        pos_off = (k % 2) * _CH

        def row_body(r, c):
            xs = []
            for j in range(8):
                sl = pl.ds(j * 16, 16)
                xs.append(rows_v[r, sl] + pos_v[pos_off + r, sl])
            acc = xs[0]
            acc2 = xs[0] * xs[0]
            for j in range(1, 8):
                acc = acc + xs[j]
                acc2 = acc2 + xs[j] * xs[j]
            s = jnp.sum(acc)
            s2 = jnp.sum(acc2)
            mean = s * _INV_D
            var = s2 * _INV_D - mean * mean
            a = _rsqrt(var + _EPS)
            nb = -mean * a
            for j in range(8):
                sl = pl.ds(j * 16, 16)
                rows_v[r, sl] = (xs[j] * a + nb) * g_v[sl] + b_v[sl]
            return c

        lax.fori_loop(0, _CH, row_body, 0)
        pltpu.sync_copy(rows_v, out_hbm.at[pl.ds(base + k * _CH, _CH)])
        return carry

    lax.fori_loop(0, _NCH, chunk_body, 0)


_sc_call = None


def _get_sc_call():
    global _sc_call
    if _sc_call is None:
        mesh = plsc.VectorSubcoreMesh(core_axis_name="c", subcore_axis_name="s")
        _sc_call = functools.partial(
            pl.kernel, mesh=mesh,
            out_type=jax.ShapeDtypeStruct((_N, _D), jnp.float32),
            scratch_types=[
                pltpu.VMEM((_NCH, _CH), jnp.int32),
                pltpu.VMEM((_CH,), jnp.int32),
                pltpu.VMEM((_L, _D), jnp.float32),
                pltpu.VMEM((_D,), jnp.float32),
                pltpu.VMEM((_D,), jnp.float32),
                pltpu.VMEM((_CH, _D), jnp.float32),
                pltpu.SemaphoreType.DMA,
            ],
        )(_body)
    return _sc_call


def kernel(input, word_table, pos_table, gamma, beta):
    idx = input.astype(jnp.int32).reshape(_NW, _NCH, _CH)
    out = _get_sc_call()(idx, word_table, pos_table, gamma, beta)
    return out.reshape(_B, _L, _D)


# speedup vs baseline: 1.3379x; 1.3379x over previous
import functools

import jax
import jax.numpy as jnp
from jax import lax
from jax.experimental import pallas as pl
from jax.experimental.pallas import tpu as pltpu
from jax.experimental.pallas import tpu_sc as plsc

_VOCAB = 100000
_L = 200
_D = 128
_B = 1024
_N = _B * _L
_NW = 32
_RPW = _N // _NW
_CH = 128
_NCH = _RPW // _CH
_EPS = 1e-12
_INV_D = 1.0 / _D


def _rsqrt(v):
    i = lax.bitcast_convert_type(v, jnp.int32)
    i = jnp.int32(0x5F3759DF) - (i >> 1)
    y = lax.bitcast_convert_type(i, jnp.float32)
    for _ in range(3):
        y = y * (1.5 - 0.5 * v * y * y)
    return y


def _body(idx_hbm, wt_hbm, pt_hbm, g_hbm, b_hbm, out_hbm,
          idx_buf, pos_v, g_v, b_v, rows_v, gsem):
    wid = lax.axis_index("s") * 2 + lax.axis_index("c")
    base = wid * _RPW

    pltpu.sync_copy(pt_hbm, pos_v)
    pltpu.sync_copy(g_hbm, g_v)
    pltpu.sync_copy(b_hbm, b_v)

    iota = lax.iota(jnp.int32, 16)
    perms = [iota ^ sh for sh in (8, 4, 2, 1)]
    _dnums = lax.GatherDimensionNumbers(
        offset_dims=(), collapsed_slice_dims=(0,), start_index_map=(0,))

    def _lane_perm(x, p):
        return lax.gather(x, p[:, None], _dnums, slice_sizes=(1,),
                          mode=lax.GatherScatterMode.PROMISE_IN_BOUNDS)

    def vsum(x):
        for p in perms:
            x = x + _lane_perm(x, p)
        return x

    def chunk_body(k, carry):
        pltpu.sync_copy(idx_hbm.at[wid, k], idx_buf)
        pltpu.async_copy(wt_hbm.at[idx_buf], rows_v, gsem).wait()
        pos_start = (k * _CH) % _L

        def row_body(r, c):
            pos_row = (pos_start + r) % _L
            xs = []
            for j in range(8):
                sl = pl.ds(j * 16, 16)
                xs.append(rows_v[r, sl] + pos_v[pos_row, sl])
            acc = xs[0]
            acc2 = xs[0] * xs[0]
            for j in range(1, 8):
                acc = acc + xs[j]
                acc2 = acc2 + xs[j] * xs[j]
            mean = vsum(acc) * _INV_D
            var = vsum(acc2) * _INV_D - mean * mean
            a = _rsqrt(var + _EPS)
            nb = -mean * a
            for j in range(8):
                sl = pl.ds(j * 16, 16)
                rows_v[r, sl] = (xs[j] * a + nb) * g_v[sl] + b_v[sl]
            return c

        lax.fori_loop(0, _CH, row_body, 0)
        pltpu.sync_copy(rows_v, out_hbm.at[pl.ds(base + k * _CH, _CH)])
        return carry

    lax.fori_loop(0, _NCH, chunk_body, 0)


_sc_call = None


def _get_sc_call():
    global _sc_call
    if _sc_call is None:
        mesh = plsc.VectorSubcoreMesh(core_axis_name="c", subcore_axis_name="s")
        _sc_call = functools.partial(
            pl.kernel, mesh=mesh,
            out_type=jax.ShapeDtypeStruct((_N, _D), jnp.float32),
            scratch_types=[
                pltpu.VMEM((_CH,), jnp.int32),
                pltpu.VMEM((_L, _D), jnp.float32),
                pltpu.VMEM((_D,), jnp.float32),
                pltpu.VMEM((_D,), jnp.float32),
                pltpu.VMEM((_CH, _D), jnp.float32),
                pltpu.SemaphoreType.DMA,
            ],
        )(_body)
    return _sc_call


def kernel(input, word_table, pos_table, gamma, beta):
    idx = input.astype(jnp.int32).reshape(_NW, _NCH, _CH)
    out = _get_sc_call()(idx, word_table, pos_table, gamma, beta)
    return out.reshape(_B, _L, _D)
